# baseline (speedup 1.0000x reference)
import functools

import jax
import jax.numpy as jnp
from jax import lax
from jax.experimental import pallas as pl
from jax.experimental.pallas import tpu as pltpu
from jax.experimental.pallas import tpu_sc as plsc

ALPHA_F, GAMMA_F = 0.25, 1.0
ALPHA_R, GAMMA_R, BETA_R = 0.5, 1.5, 0.11
VAR0, VAR1 = 0.1, 0.2
_B = 2.718281828459045 ** (GAMMA_R / ALPHA_R) - 1.0
_W = 8832


def _match_kernel(priors_ref, targets_ref,
                  gidx_ref, lidx_ref, g_ref, posf_ref, ignf_ref):
    W = priors_ref.shape[1]
    n_obj = targets_ref.shape[1]
    C = 80
    P = 8732
    img = pl.program_id(0)
    big = jnp.int32(2 ** 30)

    pcx = priors_ref[0:1, :]
    pcy = priors_ref[1:2, :]
    pw = priors_ref[2:3, :]
    ph = priors_ref[3:4, :]
    px1 = pcx - pw / 2.0
    py1 = pcy - ph / 2.0
    px2 = pcx + pw / 2.0
    py2 = pcy + ph / 2.0

    tx1 = targets_ref[0, :, 0:1]
    ty1 = targets_ref[0, :, 1:2]
    tx2 = targets_ref[0, :, 2:3]
    ty2 = targets_ref[0, :, 3:4]
    tlab = targets_ref[0, :, 4:5]

    iw = jnp.clip(jnp.minimum(tx2, px2) - jnp.maximum(tx1, px1), 0.0, None)
    ih = jnp.clip(jnp.minimum(ty2, py2) - jnp.maximum(ty1, py1), 0.0, None)
    inter = iw * ih
    area_t = (tx2 - tx1) * (ty2 - ty1)
    area_p = (px2 - px1) * (py2 - py1)
    ov = inter / (area_t + area_p - inter)

    iota_t = lax.broadcasted_iota(jnp.int32, (n_obj, W), 0)
    iota_p = lax.broadcasted_iota(jnp.int32, (n_obj, W), 1)

    bto = jnp.max(ov, axis=0, keepdims=True)
    bti = jnp.min(jnp.where(ov == bto, iota_t, big), axis=0, keepdims=True)
    bpo = jnp.max(ov, axis=1, keepdims=True)
    bpi = jnp.min(jnp.where(ov == bpo, iota_p, big), axis=1, keepdims=True)

    forced_t = jnp.max(jnp.where(iota_p == bpi, iota_t, -1),
                       axis=0, keepdims=True)
    forced = forced_t >= 0
    bto = jnp.where(forced, 2.0, bto)
    bti = jnp.where(forced, forced_t, bti)

    eq = (bti == iota_t).astype(jnp.float32)
    mx1 = jnp.sum(eq * tx1, axis=0, keepdims=True)
    my1 = jnp.sum(eq * ty1, axis=0, keepdims=True)
    mx2 = jnp.sum(eq * tx2, axis=0, keepdims=True)
    my2 = jnp.sum(eq * ty2, axis=0, keepdims=True)
    mlab = jnp.sum(eq * tlab, axis=0, keepdims=True)

    posf = (bto >= 0.5).astype(jnp.float32)
    ignf = jnp.logical_and(bto >= 0.4, bto < 0.5).astype(jnp.float32)
    cls = mlab.astype(jnp.int32)

    ip = lax.broadcasted_iota(jnp.int32, (1, W), 1)
    ipc = jnp.minimum(ip, P - 1)
    gidx_ref[0] = (img * P + ipc) * C + cls
    lbase = (img * P + ipc) * 4
    lidx_ref[0] = jnp.concatenate(
        [lbase, lbase + 1, lbase + 2, lbase + 3], axis=0)
    posf_ref[0] = posf
    ignf_ref[0] = ignf

    gcx = ((mx1 + mx2) / 2.0 - pcx) / (VAR0 * pw)
    gcy = ((my1 + my2) / 2.0 - pcy) / (VAR0 * ph)
    gw = jnp.log((mx2 - mx1) / pw) / VAR1
    gh = jnp.log((my2 - my1) / ph) / VAR1
    g_ref[0] = jnp.concatenate([gcx, gcy, gw, gh], axis=0)


def _f0_terms(x):
    u = jnp.exp(-jnp.abs(x))
    sp = jnp.maximum(x, 0.0) + jnp.log1p(u)
    r = 1.0 / (1.0 + u)
    sig = jnp.where(x >= 0.0, r, u * r)
    return sp, sig


def _bulk_kernel(conf_ref, out_ref):
    j = pl.program_id(0)

    @pl.when(j == 0)
    def _init():
        out_ref[...] = jnp.zeros_like(out_ref)

    sp, sig = _f0_terms(conf_ref[...])
    out_ref[...] += (1.0 - ALPHA_F) * jnp.sum(sp * sig, axis=(0, 1),
                                              keepdims=True)


def _loss_kernel(xl_ref, g_ref, xg_ref, posf_ref, ignf_ref,
                 loc_sum_ref, corr_ref, pos_cnt_ref, ign_cnt_ref):
    posf = posf_ref[0]
    ignf = ignf_ref[0]

    d = jnp.abs(xl_ref[0] - g_ref[0])
    small = (ALPHA_R / _B * (_B * d + 1.0)
             * jnp.log(_B * d / BETA_R + 1.0) - ALPHA_R * d)
    large = GAMMA_R * d + GAMMA_R / _B - ALPHA_R * BETA_R
    bl = jnp.where(d < BETA_R, small, large)
    loc_sum_ref[0] = jnp.sum(bl * posf, axis=(0, 1), keepdims=True)
    pos_cnt_ref[0] = jnp.sum(posf, axis=(0, 1), keepdims=True)
    ign_cnt_ref[0] = jnp.sum(ignf, axis=(0, 1), keepdims=True)

    x = xg_ref[0]
    sp, sig = _f0_terms(x)
    f0 = (1.0 - ALPHA_F) * sp * sig
    f1 = ALPHA_F * (sp - x) * (1.0 - sig)
    corr = posf * (f1 - f0) - ignf * f0
    corr_ref[0] = jnp.sum(corr, axis=(0, 1), keepdims=True)


def _sc_gather2(conf_flat, gidx_flat, loc_flat, lidx_flat):
    info = plsc.get_sparse_core_info()
    nc = info.num_cores
    nw = nc * info.num_subcores
    b1 = gidx_flat.shape[0]
    b2 = lidx_flat.shape[0]
    w1 = b1 // nw
    w2 = b2 // nw
    mesh = plsc.VectorSubcoreMesh(core_axis_name="c", subcore_axis_name="s")

    @functools.partial(
        pl.kernel, mesh=mesh,
        out_type=(jax.ShapeDtypeStruct((b1,), jnp.float32),
                  jax.ShapeDtypeStruct((b2,), jnp.float32)),
        scratch_types=[
            pltpu.VMEM((w1,), jnp.int32),
            pltpu.VMEM((w1,), jnp.float32),
            pltpu.VMEM((w2,), jnp.int32),
            pltpu.VMEM((w2,), jnp.float32),
            pltpu.SemaphoreType.DMA,
            pltpu.SemaphoreType.DMA,
        ],
    )
    def gather_k(conf_hbm, gidx_hbm, loc_hbm, lidx_hbm, xg_hbm, xl_hbm,
                 gi_v, gr_v, li_v, lr_v, sem1, sem2):
        wid = lax.axis_index("s") * nc + lax.axis_index("c")
        pltpu.sync_copy(gidx_hbm.at[pl.ds(wid * w1, w1)], gi_v)
        pltpu.sync_copy(lidx_hbm.at[pl.ds(wid * w2, w2)], li_v)
        c1 = pltpu.async_copy(conf_hbm.at[gi_v], gr_v, sem1)
        c2 = pltpu.async_copy(loc_hbm.at[li_v], lr_v, sem2)
        c1.wait()
        c2.wait()
        pltpu.sync_copy(gr_v, xg_hbm.at[pl.ds(wid * w1, w1)])
        pltpu.sync_copy(lr_v, xl_hbm.at[pl.ds(wid * w2, w2)])

    return gather_k(conf_flat, gidx_flat, loc_flat, lidx_flat)


@jax.jit
def kernel(loc_data, conf_data, priors, targets):
    num, num_priors, num_classes = conf_data.shape
    P = num_priors
    n_obj = targets.shape[1]

    pad = jnp.tile(jnp.array([[-100.0, -100.0, 1.0, 1.0]], jnp.float32),
                   (_W - P, 1))
    priors_p = jnp.transpose(jnp.concatenate([priors, pad], axis=0), (1, 0))

    vi_sd = jax.ShapeDtypeStruct((num, 1, _W), jnp.int32)
    vf_sd = jax.ShapeDtypeStruct((num, 1, _W), jnp.float32)
    gidx, lidx, g, posf, ignf = pl.pallas_call(
        _match_kernel,
        grid=(num,),
        in_specs=[
            pl.BlockSpec((4, _W), lambda i: (0, 0)),
            pl.BlockSpec((1, n_obj, 5), lambda i: (i, 0, 0)),
        ],
        out_specs=[
            pl.BlockSpec((1, 1, _W), lambda i: (i, 0, 0)),
            pl.BlockSpec((1, 4, _W), lambda i: (i, 0, 0)),
            pl.BlockSpec((1, 4, _W), lambda i: (i, 0, 0)),
            pl.BlockSpec((1, 1, _W), lambda i: (i, 0, 0)),
            pl.BlockSpec((1, 1, _W), lambda i: (i, 0, 0)),
        ],
        out_shape=[vi_sd,
                   jax.ShapeDtypeStruct((num, 4, _W), jnp.int32),
                   jax.ShapeDtypeStruct((num, 4, _W), jnp.float32),
                   vf_sd, vf_sd],
        compiler_params=pltpu.CompilerParams(
            dimension_semantics=("arbitrary",),
        ),
    )(priors_p, targets)

    xg, xl = _sc_gather2(conf_data.reshape(-1), gidx.reshape(-1),
                         loc_data.reshape(-1), lidx.reshape(-1))

    total = num * P * num_classes
    conf_flat2 = conf_data.reshape(total // 1280, 1280)
    rows = conf_flat2.shape[0]
    blk = 472
    s0 = pl.pallas_call(
        _bulk_kernel,
        grid=(rows // blk,),
        in_specs=[pl.BlockSpec((blk, 1280), lambda j: (j, 0))],
        out_specs=pl.BlockSpec((1, 1), lambda j: (0, 0)),
        out_shape=jax.ShapeDtypeStruct((1, 1), jnp.float32),
        compiler_params=pltpu.CompilerParams(
            dimension_semantics=("arbitrary",),
        ),
    )(conf_flat2)

    sc_sd = jax.ShapeDtypeStruct((num, 1, 1), jnp.float32)
    loc_sum, corr, pos_cnt, ign_cnt = pl.pallas_call(
        _loss_kernel,
        grid=(num,),
        in_specs=[
            pl.BlockSpec((1, 4, _W), lambda i: (i, 0, 0)),
            pl.BlockSpec((1, 4, _W), lambda i: (i, 0, 0)),
            pl.BlockSpec((1, 1, _W), lambda i: (i, 0, 0)),
            pl.BlockSpec((1, 1, _W), lambda i: (i, 0, 0)),
            pl.BlockSpec((1, 1, _W), lambda i: (i, 0, 0)),
        ],
        out_specs=[pl.BlockSpec((1, 1, 1), lambda i: (i, 0, 0))] * 4,
        out_shape=[sc_sd] * 4,
        compiler_params=pltpu.CompilerParams(
            dimension_semantics=("arbitrary",),
        ),
    )(xl.reshape(num, 4, _W), g, xg.reshape(num, 1, _W), posf, ignf)

    loss_l = jnp.sum(loc_sum) / (4.0 * jnp.sum(pos_cnt))
    denom = jnp.float32(total) - jnp.sum(ign_cnt)
    loss_c = (s0[0, 0] + jnp.sum(corr)) / denom
    return (loss_l, loss_c)


# speedup vs baseline: 5.9972x; 5.9972x over previous
import functools

import jax
import jax.numpy as jnp
from jax import lax
from jax.experimental import pallas as pl
from jax.experimental.pallas import tpu as pltpu
from jax.experimental.pallas import tpu_sc as plsc

ALPHA_F, GAMMA_F = 0.25, 1.0
ALPHA_R, GAMMA_R, BETA_R = 0.5, 1.5, 0.11
VAR0, VAR1 = 0.1, 0.2
_B = 2.718281828459045 ** (GAMMA_R / ALPHA_R) - 1.0
_W = 8832


def _match_kernel(priors_ref, targets_ref,
                  gidx_ref, g_ref, posf_ref, ignf_ref):
    W = priors_ref.shape[1]
    n_obj = targets_ref.shape[1]
    C = 80
    P = 8732
    img = pl.program_id(0)
    big = jnp.int32(2 ** 30)

    pcx = priors_ref[0:1, :]
    pcy = priors_ref[1:2, :]
    pw = priors_ref[2:3, :]
    ph = priors_ref[3:4, :]
    px1 = pcx - pw / 2.0
    py1 = pcy - ph / 2.0
    px2 = pcx + pw / 2.0
    py2 = pcy + ph / 2.0

    tx1 = targets_ref[0, :, 0:1]
    ty1 = targets_ref[0, :, 1:2]
    tx2 = targets_ref[0, :, 2:3]
    ty2 = targets_ref[0, :, 3:4]
    tlab = targets_ref[0, :, 4:5]

    iw = jnp.clip(jnp.minimum(tx2, px2) - jnp.maximum(tx1, px1), 0.0, None)
    ih = jnp.clip(jnp.minimum(ty2, py2) - jnp.maximum(ty1, py1), 0.0, None)
    inter = iw * ih
    area_t = (tx2 - tx1) * (ty2 - ty1)
    area_p = (px2 - px1) * (py2 - py1)
    ov = inter / (area_t + area_p - inter)

    iota_t = lax.broadcasted_iota(jnp.int32, (n_obj, W), 0)
    iota_p = lax.broadcasted_iota(jnp.int32, (n_obj, W), 1)

    bto = jnp.max(ov, axis=0, keepdims=True)
    bti = jnp.min(jnp.where(ov == bto, iota_t, big), axis=0, keepdims=True)
    bpo = jnp.max(ov, axis=1, keepdims=True)
    bpi = jnp.min(jnp.where(ov == bpo, iota_p, big), axis=1, keepdims=True)

    forced_t = jnp.max(jnp.where(iota_p == bpi, iota_t, -1),
                       axis=0, keepdims=True)
    forced = forced_t >= 0
    bto = jnp.where(forced, 2.0, bto)
    bti = jnp.where(forced, forced_t, bti)

    eq = (bti == iota_t).astype(jnp.float32)
    mx1 = jnp.sum(eq * tx1, axis=0, keepdims=True)
    my1 = jnp.sum(eq * ty1, axis=0, keepdims=True)
    mx2 = jnp.sum(eq * tx2, axis=0, keepdims=True)
    my2 = jnp.sum(eq * ty2, axis=0, keepdims=True)
    mlab = jnp.sum(eq * tlab, axis=0, keepdims=True)

    posf = (bto >= 0.5).astype(jnp.float32)
    ignf = jnp.logical_and(bto >= 0.4, bto < 0.5).astype(jnp.float32)
    cls = mlab.astype(jnp.int32)

    ip = lax.broadcasted_iota(jnp.int32, (1, W), 1)
    ipc = jnp.minimum(ip, P - 1)
    gidx_ref[0] = (img * C + cls) * P + ipc
    posf_ref[0] = posf
    ignf_ref[0] = ignf

    gcx = ((mx1 + mx2) / 2.0 - pcx) / (VAR0 * pw)
    gcy = ((my1 + my2) / 2.0 - pcy) / (VAR0 * ph)
    gw = jnp.log((mx2 - mx1) / pw) / VAR1
    gh = jnp.log((my2 - my1) / ph) / VAR1
    g_ref[0] = jnp.concatenate([gcx, gcy, gw, gh], axis=0)


def _f0_terms(x):
    u = jnp.exp(-jnp.abs(x))
    sp = jnp.maximum(x, 0.0) + jnp.log1p(u)
    r = 1.0 / (1.0 + u)
    sig = jnp.where(x >= 0.0, r, u * r)
    return sp, sig


def _bulk_kernel(conf_ref, out_ref):
    j = pl.program_id(0)

    @pl.when(j == 0)
    def _init():
        out_ref[...] = jnp.zeros_like(out_ref)

    sp, sig = _f0_terms(conf_ref[...])
    out_ref[...] += (1.0 - ALPHA_F) * jnp.sum(sp * sig, axis=(0, 1),
                                              keepdims=True)


def _loss_kernel(loc_ref, g_ref, xg_ref, posf_ref, ignf_ref,
                 loc_sum_ref, corr_ref, pos_cnt_ref, ign_cnt_ref):
    P = loc_ref.shape[2]
    posf = posf_ref[0]
    ignf = ignf_ref[0]

    d = jnp.abs(loc_ref[0] - g_ref[0][:, :P])
    small = (ALPHA_R / _B * (_B * d + 1.0)
             * jnp.log(_B * d / BETA_R + 1.0) - ALPHA_R * d)
    large = GAMMA_R * d + GAMMA_R / _B - ALPHA_R * BETA_R
    bl = jnp.where(d < BETA_R, small, large)
    loc_sum_ref[0] = jnp.sum(bl * posf[:, :P], axis=(0, 1), keepdims=True)
    pos_cnt_ref[0] = jnp.sum(posf, axis=(0, 1), keepdims=True)
    ign_cnt_ref[0] = jnp.sum(ignf, axis=(0, 1), keepdims=True)

    x = xg_ref[0]
    sp, sig = _f0_terms(x)
    f0 = (1.0 - ALPHA_F) * sp * sig
    f1 = ALPHA_F * (sp - x) * (1.0 - sig)
    corr = posf * (f1 - f0) - ignf * f0
    corr_ref[0] = jnp.sum(corr, axis=(0, 1), keepdims=True)


def _sc_gather(conf_flat, gidx_flat):
    info = plsc.get_sparse_core_info()
    nc = info.num_cores
    nw = nc * info.num_subcores
    b = gidx_flat.shape[0]
    w = b // nw
    mesh = plsc.VectorSubcoreMesh(core_axis_name="c", subcore_axis_name="s")

    @functools.partial(
        pl.kernel, mesh=mesh,
        out_type=jax.ShapeDtypeStruct((b,), jnp.float32),
        scratch_types=[
            pltpu.VMEM((w,), jnp.int32),
            pltpu.VMEM((w,), jnp.float32),
            pltpu.SemaphoreType.DMA,
        ],
    )
    def gather_k(conf_hbm, gidx_hbm, out_hbm, gi_v, gr_v, sem):
        wid = lax.axis_index("s") * nc + lax.axis_index("c")
        pltpu.sync_copy(gidx_hbm.at[pl.ds(wid * w, w)], gi_v)
        pltpu.async_copy(conf_hbm.at[gi_v], gr_v, sem).wait()
        pltpu.sync_copy(gr_v, out_hbm.at[pl.ds(wid * w, w)])

    return gather_k(conf_flat, gidx_flat)


@jax.jit
def kernel(loc_data, conf_data, priors, targets):
    num, num_priors, num_classes = conf_data.shape
    P = num_priors
    n_obj = targets.shape[1]

    priors_t = jnp.transpose(priors, (1, 0))
    loc_t = jnp.transpose(loc_data, (0, 2, 1))
    conf_t = jnp.transpose(conf_data, (0, 2, 1))

    padc = jnp.array([[-100.0], [-100.0], [1.0], [1.0]], jnp.float32)
    priors_p = jnp.concatenate(
        [priors_t, jnp.tile(padc, (1, _W - P))], axis=1)

    vi_sd = jax.ShapeDtypeStruct((num, 1, _W), jnp.int32)
    vf_sd = jax.ShapeDtypeStruct((num, 1, _W), jnp.float32)
    gidx, g, posf, ignf = pl.pallas_call(
        _match_kernel,
        grid=(num,),
        in_specs=[
            pl.BlockSpec((4, _W), lambda i: (0, 0)),
            pl.BlockSpec((1, n_obj, 5), lambda i: (i, 0, 0)),
        ],
        out_specs=[
            pl.BlockSpec((1, 1, _W), lambda i: (i, 0, 0)),
            pl.BlockSpec((1, 4, _W), lambda i: (i, 0, 0)),
            pl.BlockSpec((1, 1, _W), lambda i: (i, 0, 0)),
            pl.BlockSpec((1, 1, _W), lambda i: (i, 0, 0)),
        ],
        out_shape=[vi_sd,
                   jax.ShapeDtypeStruct((num, 4, _W), jnp.float32),
                   vf_sd, vf_sd],
        compiler_params=pltpu.CompilerParams(
            dimension_semantics=("arbitrary",),
        ),
    )(priors_p, targets)

    xg = _sc_gather(conf_t.reshape(-1), gidx.reshape(-1))

    total = num * P * num_classes
    conf_flat2 = conf_t.reshape(total // 1280, 1280)
    rows = conf_flat2.shape[0]
    blk = 472
    s0 = pl.pallas_call(
        _bulk_kernel,
        grid=(rows // blk,),
        in_specs=[pl.BlockSpec((blk, 1280), lambda j: (j, 0))],
        out_specs=pl.BlockSpec((1, 1), lambda j: (0, 0)),
        out_shape=jax.ShapeDtypeStruct((1, 1), jnp.float32),
        compiler_params=pltpu.CompilerParams(
            dimension_semantics=("arbitrary",),
        ),
    )(conf_flat2)

    sc_sd = jax.ShapeDtypeStruct((num, 1, 1), jnp.float32)
    loc_sum, corr, pos_cnt, ign_cnt = pl.pallas_call(
        _loss_kernel,
        grid=(num,),
        in_specs=[
            pl.BlockSpec((1, 4, P), lambda i: (i, 0, 0)),
            pl.BlockSpec((1, 4, _W), lambda i: (i, 0, 0)),
            pl.BlockSpec((1, 1, _W), lambda i: (i, 0, 0)),
            pl.BlockSpec((1, 1, _W), lambda i: (i, 0, 0)),
            pl.BlockSpec((1, 1, _W), lambda i: (i, 0, 0)),
        ],
        out_specs=[pl.BlockSpec((1, 1, 1), lambda i: (i, 0, 0))] * 4,
        out_shape=[sc_sd] * 4,
        compiler_params=pltpu.CompilerParams(
            dimension_semantics=("arbitrary",),
        ),
    )(loc_t, g, xg.reshape(num, 1, _W), posf, ignf)

    loss_l = jnp.sum(loc_sum) / (4.0 * jnp.sum(pos_cnt))
    denom = jnp.float32(total) - jnp.sum(ign_cnt)
    loss_c = (s0[0, 0] + jnp.sum(corr)) / denom
    return (loss_l, loss_c)


# speedup vs baseline: 7.4464x; 1.2416x over previous
import functools

import jax
import jax.numpy as jnp
from jax import lax
from jax.experimental import pallas as pl
from jax.experimental.pallas import tpu as pltpu
from jax.experimental.pallas import tpu_sc as plsc

ALPHA_F, GAMMA_F = 0.25, 1.0
ALPHA_R, GAMMA_R, BETA_R = 0.5, 1.5, 0.11
VAR0, VAR1 = 0.1, 0.2
_B = 2.718281828459045 ** (GAMMA_R / ALPHA_R) - 1.0
_W = 8832


def _match_kernel(priors_ref, targets_ref,
                  gidx_ref, g_ref, posf_ref, ignf_ref):
    W = priors_ref.shape[1]
    n_obj = targets_ref.shape[1]
    C = 80
    P = 8732
    img = pl.program_id(0)
    big = jnp.int32(2 ** 30)

    pcx = priors_ref[0:1, :]
    pcy = priors_ref[1:2, :]
    pw = priors_ref[2:3, :]
    ph = priors_ref[3:4, :]
    px1 = pcx - pw / 2.0
    py1 = pcy - ph / 2.0
    px2 = pcx + pw / 2.0
    py2 = pcy + ph / 2.0

    tx1 = targets_ref[0, :, 0:1]
    ty1 = targets_ref[0, :, 1:2]
    tx2 = targets_ref[0, :, 2:3]
    ty2 = targets_ref[0, :, 3:4]
    tlab = targets_ref[0, :, 4:5]

    iw = jnp.clip(jnp.minimum(tx2, px2) - jnp.maximum(tx1, px1), 0.0, None)
    ih = jnp.clip(jnp.minimum(ty2, py2) - jnp.maximum(ty1, py1), 0.0, None)
    inter = iw * ih
    area_t = (tx2 - tx1) * (ty2 - ty1)
    area_p = (px2 - px1) * (py2 - py1)
    ov = inter / (area_t + area_p - inter)

    iota_t = lax.broadcasted_iota(jnp.int32, (n_obj, W), 0)
    iota_p = lax.broadcasted_iota(jnp.int32, (n_obj, W), 1)

    bto = jnp.max(ov, axis=0, keepdims=True)
    bti = jnp.min(jnp.where(ov == bto, iota_t, big), axis=0, keepdims=True)
    bpo = jnp.max(ov, axis=1, keepdims=True)
    bpi = jnp.min(jnp.where(ov == bpo, iota_p, big), axis=1, keepdims=True)

    forced_t = jnp.max(jnp.where(iota_p == bpi, iota_t, -1),
                       axis=0, keepdims=True)
    forced = forced_t >= 0
    bto = jnp.where(forced, 2.0, bto)
    bti = jnp.where(forced, forced_t, bti)

    eq = (bti == iota_t).astype(jnp.float32)
    mx1 = jnp.sum(eq * tx1, axis=0, keepdims=True)
    my1 = jnp.sum(eq * ty1, axis=0, keepdims=True)
    mx2 = jnp.sum(eq * tx2, axis=0, keepdims=True)
    my2 = jnp.sum(eq * ty2, axis=0, keepdims=True)
    mlab = jnp.sum(eq * tlab, axis=0, keepdims=True)

    posf = (bto >= 0.5).astype(jnp.float32)
    ignf = jnp.logical_and(bto >= 0.4, bto < 0.5).astype(jnp.float32)
    cls = mlab.astype(jnp.int32)

    ip = lax.broadcasted_iota(jnp.int32, (1, W), 1)
    ipc = jnp.minimum(ip, P - 1)
    gidx_ref[0] = (img * C + cls) * P + ipc
    posf_ref[0] = posf
    ignf_ref[0] = ignf

    gcx = ((mx1 + mx2) / 2.0 - pcx) / (VAR0 * pw)
    gcy = ((my1 + my2) / 2.0 - pcy) / (VAR0 * ph)
    gw = jnp.log((mx2 - mx1) / pw) / VAR1
    gh = jnp.log((my2 - my1) / ph) / VAR1
    g_ref[0] = jnp.concatenate([gcx, gcy, gw, gh], axis=0)


def _f0_terms(x):
    u = jnp.exp(-jnp.abs(x))
    sp = jnp.maximum(x, 0.0) + jnp.log1p(u)
    r = 1.0 / (1.0 + u)
    sig = jnp.where(x >= 0.0, r, u * r)
    return sp, sig


def _loss_kernel(conf_ref, loc_ref, g_ref, xg_ref, posf_ref, ignf_ref,
                 s0_ref, loc_sum_ref, corr_ref, pos_cnt_ref, ign_cnt_ref):
    P = loc_ref.shape[2]
    posf = posf_ref[0]
    ignf = ignf_ref[0]

    spb, sigb = _f0_terms(conf_ref[0])
    s0_ref[0] = (1.0 - ALPHA_F) * jnp.sum(spb * sigb, axis=(0, 1),
                                          keepdims=True)

    d = jnp.abs(loc_ref[0] - g_ref[0][:, :P])
    small = (ALPHA_R / _B * (_B * d + 1.0)
             * jnp.log(_B * d / BETA_R + 1.0) - ALPHA_R * d)
    large = GAMMA_R * d + GAMMA_R / _B - ALPHA_R * BETA_R
    bl = jnp.where(d < BETA_R, small, large)
    loc_sum_ref[0] = jnp.sum(bl * posf[:, :P], axis=(0, 1), keepdims=True)
    pos_cnt_ref[0] = jnp.sum(posf, axis=(0, 1), keepdims=True)
    ign_cnt_ref[0] = jnp.sum(ignf, axis=(0, 1), keepdims=True)

    x = xg_ref[0]
    sp, sig = _f0_terms(x)
    f0 = (1.0 - ALPHA_F) * sp * sig
    f1 = ALPHA_F * (sp - x) * (1.0 - sig)
    corr = posf * (f1 - f0) - ignf * f0
    corr_ref[0] = jnp.sum(corr, axis=(0, 1), keepdims=True)


def _sc_gather(conf_flat, gidx_flat):
    info = plsc.get_sparse_core_info()
    nc = info.num_cores
    nw = nc * info.num_subcores
    b = gidx_flat.shape[0]
    w = b // nw
    mesh = plsc.VectorSubcoreMesh(core_axis_name="c", subcore_axis_name="s")

    @functools.partial(
        pl.kernel, mesh=mesh,
        out_type=jax.ShapeDtypeStruct((b,), jnp.float32),
        scratch_types=[
            pltpu.VMEM((w,), jnp.int32),
            pltpu.VMEM((w,), jnp.float32),
            pltpu.SemaphoreType.DMA,
        ],
    )
    def gather_k(conf_hbm, gidx_hbm, out_hbm, gi_v, gr_v, sem):
        wid = lax.axis_index("s") * nc + lax.axis_index("c")
        pltpu.sync_copy(gidx_hbm.at[pl.ds(wid * w, w)], gi_v)
        pltpu.async_copy(conf_hbm.at[gi_v], gr_v, sem).wait()
        pltpu.sync_copy(gr_v, out_hbm.at[pl.ds(wid * w, w)])

    return gather_k(conf_flat, gidx_flat)


@jax.jit
def kernel(loc_data, conf_data, priors, targets):
    num, num_priors, num_classes = conf_data.shape
    P = num_priors
    n_obj = targets.shape[1]

    priors_t = jnp.transpose(priors, (1, 0))
    loc_t = jnp.transpose(loc_data, (0, 2, 1))
    conf_t = jnp.transpose(conf_data, (0, 2, 1))

    padc = jnp.array([[-100.0], [-100.0], [1.0], [1.0]], jnp.float32)
    priors_p = jnp.concatenate(
        [priors_t, jnp.tile(padc, (1, _W - P))], axis=1)

    vi_sd = jax.ShapeDtypeStruct((num, 1, _W), jnp.int32)
    vf_sd = jax.ShapeDtypeStruct((num, 1, _W), jnp.float32)
    gidx, g, posf, ignf = pl.pallas_call(
        _match_kernel,
        grid=(num,),
        in_specs=[
            pl.BlockSpec((4, _W), lambda i: (0, 0)),
            pl.BlockSpec((1, n_obj, 5), lambda i: (i, 0, 0)),
        ],
        out_specs=[
            pl.BlockSpec((1, 1, _W), lambda i: (i, 0, 0)),
            pl.BlockSpec((1, 4, _W), lambda i: (i, 0, 0)),
            pl.BlockSpec((1, 1, _W), lambda i: (i, 0, 0)),
            pl.BlockSpec((1, 1, _W), lambda i: (i, 0, 0)),
        ],
        out_shape=[vi_sd,
                   jax.ShapeDtypeStruct((num, 4, _W), jnp.float32),
                   vf_sd, vf_sd],
        compiler_params=pltpu.CompilerParams(
            dimension_semantics=("arbitrary",),
        ),
    )(priors_p, targets)

    xg = _sc_gather(conf_t.reshape(-1), gidx.reshape(-1))

    total = num * P * num_classes
    sc_sd = jax.ShapeDtypeStruct((num, 1, 1), jnp.float32)
    s0, loc_sum, corr, pos_cnt, ign_cnt = pl.pallas_call(
        _loss_kernel,
        grid=(num,),
        in_specs=[
            pl.BlockSpec((1, num_classes, P), lambda i: (i, 0, 0)),
            pl.BlockSpec((1, 4, P), lambda i: (i, 0, 0)),
            pl.BlockSpec((1, 4, _W), lambda i: (i, 0, 0)),
            pl.BlockSpec((1, 1, _W), lambda i: (i, 0, 0)),
            pl.BlockSpec((1, 1, _W), lambda i: (i, 0, 0)),
            pl.BlockSpec((1, 1, _W), lambda i: (i, 0, 0)),
        ],
        out_specs=[pl.BlockSpec((1, 1, 1), lambda i: (i, 0, 0))] * 5,
        out_shape=[sc_sd] * 5,
        compiler_params=pltpu.CompilerParams(
            dimension_semantics=("arbitrary",),
        ),
    )(conf_t, loc_t, g, xg.reshape(num, 1, _W), posf, ignf)

    loss_l = jnp.sum(loc_sum) / (4.0 * jnp.sum(pos_cnt))
    denom = jnp.float32(total) - jnp.sum(ign_cnt)
    loss_c = (jnp.sum(s0) + jnp.sum(corr)) / denom
    return (loss_l, loss_c)


# speedup vs baseline: 7.7573x; 1.0418x over previous
import functools

import jax
import jax.numpy as jnp
from jax import lax
from jax.experimental import pallas as pl
from jax.experimental.pallas import tpu as pltpu
from jax.experimental.pallas import tpu_sc as plsc

ALPHA_F, GAMMA_F = 0.25, 1.0
ALPHA_R, GAMMA_R, BETA_R = 0.5, 1.5, 0.11
VAR0, VAR1 = 0.1, 0.2
_B = 2.718281828459045 ** (GAMMA_R / ALPHA_R) - 1.0
_W = 8832


def _match_kernel(priors_ref, targets_ref,
                  gidx_ref, g_ref, posf_ref, ignf_ref):
    W = priors_ref.shape[1]
    n_obj = targets_ref.shape[1]
    nb = targets_ref.shape[0]
    C = 80
    P = 8732
    big = jnp.int32(2 ** 30)

    pcx = priors_ref[0:1, :]
    pcy = priors_ref[1:2, :]
    pw = priors_ref[2:3, :]
    ph = priors_ref[3:4, :]
    px1 = pcx - pw / 2.0
    py1 = pcy - ph / 2.0
    px2 = pcx + pw / 2.0
    py2 = pcy + ph / 2.0

    iota_t = lax.broadcasted_iota(jnp.int32, (n_obj, W), 0)
    iota_p = lax.broadcasted_iota(jnp.int32, (n_obj, W), 1)
    ip = lax.broadcasted_iota(jnp.int32, (1, W), 1)
    ipc = jnp.minimum(ip, P - 1)

    for b in range(nb):
        img = pl.program_id(0) * nb + b
        tx1 = targets_ref[b, :, 0:1]
        ty1 = targets_ref[b, :, 1:2]
        tx2 = targets_ref[b, :, 2:3]
        ty2 = targets_ref[b, :, 3:4]
        tlab = targets_ref[b, :, 4:5]

        iw = jnp.clip(jnp.minimum(tx2, px2) - jnp.maximum(tx1, px1), 0.0, None)
        ih = jnp.clip(jnp.minimum(ty2, py2) - jnp.maximum(ty1, py1), 0.0, None)
        inter = iw * ih
        area_t = (tx2 - tx1) * (ty2 - ty1)
        area_p = (px2 - px1) * (py2 - py1)
        ov = inter / (area_t + area_p - inter)

        bto = jnp.max(ov, axis=0, keepdims=True)
        bti = jnp.min(jnp.where(ov == bto, iota_t, big), axis=0,
                      keepdims=True)
        bpo = jnp.max(ov, axis=1, keepdims=True)
        bpi = jnp.min(jnp.where(ov == bpo, iota_p, big), axis=1,
                      keepdims=True)

        forced_t = jnp.max(jnp.where(iota_p == bpi, iota_t, -1),
                           axis=0, keepdims=True)
        forced = forced_t >= 0
        bto = jnp.where(forced, 2.0, bto)
        bti = jnp.where(forced, forced_t, bti)

        eq = (bti == iota_t).astype(jnp.float32)
        mx1 = jnp.sum(eq * tx1, axis=0, keepdims=True)
        my1 = jnp.sum(eq * ty1, axis=0, keepdims=True)
        mx2 = jnp.sum(eq * tx2, axis=0, keepdims=True)
        my2 = jnp.sum(eq * ty2, axis=0, keepdims=True)
        mlab = jnp.sum(eq * tlab, axis=0, keepdims=True)

        posf = (bto >= 0.5).astype(jnp.float32)
        ignf = jnp.logical_and(bto >= 0.4, bto < 0.5).astype(jnp.float32)
        cls = mlab.astype(jnp.int32)

        gidx_ref[b] = (img * C + cls) * P + ipc
        posf_ref[b] = posf
        ignf_ref[b] = ignf

        gcx = ((mx1 + mx2) / 2.0 - pcx) / (VAR0 * pw)
        gcy = ((my1 + my2) / 2.0 - pcy) / (VAR0 * ph)
        gw = jnp.log((mx2 - mx1) / pw) / VAR1
        gh = jnp.log((my2 - my1) / ph) / VAR1
        g_ref[b] = jnp.concatenate([gcx, gcy, gw, gh], axis=0)


def _f0_terms(x):
    u = jnp.exp(-jnp.abs(x))
    sp = jnp.maximum(x, 0.0) + jnp.log1p(u)
    r = 1.0 / (1.0 + u)
    sig = jnp.where(x >= 0.0, r, u * r)
    return sp, sig


def _loss_kernel(conf_ref, loc_ref, g_ref, xg_ref, posf_ref, ignf_ref,
                 s0_ref, loc_sum_ref, corr_ref, pos_cnt_ref, ign_cnt_ref):
    P = loc_ref.shape[2]
    nb = loc_ref.shape[0]
    for b in range(nb):
        posf = posf_ref[b]
        ignf = ignf_ref[b]

        spb, sigb = _f0_terms(conf_ref[b])
        s0_ref[b] = (1.0 - ALPHA_F) * jnp.sum(spb * sigb, axis=(0, 1),
                                              keepdims=True)

        d = jnp.abs(loc_ref[b] - g_ref[b][:, :P])
        small = (ALPHA_R / _B * (_B * d + 1.0)
                 * jnp.log(_B * d / BETA_R + 1.0) - ALPHA_R * d)
        large = GAMMA_R * d + GAMMA_R / _B - ALPHA_R * BETA_R
        bl = jnp.where(d < BETA_R, small, large)
        loc_sum_ref[b] = jnp.sum(bl * posf[:, :P], axis=(0, 1),
                                 keepdims=True)
        pos_cnt_ref[b] = jnp.sum(posf, axis=(0, 1), keepdims=True)
        ign_cnt_ref[b] = jnp.sum(ignf, axis=(0, 1), keepdims=True)

        x = xg_ref[b]
        sp, sig = _f0_terms(x)
        f0 = (1.0 - ALPHA_F) * sp * sig
        f1 = ALPHA_F * (sp - x) * (1.0 - sig)
        corr = posf * (f1 - f0) - ignf * f0
        corr_ref[b] = jnp.sum(corr, axis=(0, 1), keepdims=True)


def _sc_gather(conf_flat, gidx_flat):
    info = plsc.get_sparse_core_info()
    nc = info.num_cores
    nw = nc * info.num_subcores
    b = gidx_flat.shape[0]
    w = b // nw
    mesh = plsc.VectorSubcoreMesh(core_axis_name="c", subcore_axis_name="s")

    @functools.partial(
        pl.kernel, mesh=mesh,
        out_type=jax.ShapeDtypeStruct((b,), jnp.float32),
        scratch_types=[
            pltpu.VMEM((w,), jnp.int32),
            pltpu.VMEM((w,), jnp.float32),
            pltpu.SemaphoreType.DMA,
        ],
    )
    def gather_k(conf_hbm, gidx_hbm, out_hbm, gi_v, gr_v, sem):
        wid = lax.axis_index("s") * nc + lax.axis_index("c")
        pltpu.sync_copy(gidx_hbm.at[pl.ds(wid * w, w)], gi_v)
        pltpu.async_copy(conf_hbm.at[gi_v], gr_v, sem).wait()
        pltpu.sync_copy(gr_v, out_hbm.at[pl.ds(wid * w, w)])

    return gather_k(conf_flat, gidx_flat)


@jax.jit
def kernel(loc_data, conf_data, priors, targets):
    num, num_priors, num_classes = conf_data.shape
    P = num_priors
    n_obj = targets.shape[1]

    priors_t = jnp.transpose(priors, (1, 0))
    loc_t = jnp.transpose(loc_data, (0, 2, 1))
    conf_t = jnp.transpose(conf_data, (0, 2, 1))

    padc = jnp.array([[-100.0], [-100.0], [1.0], [1.0]], jnp.float32)
    priors_p = jnp.concatenate(
        [priors_t, jnp.tile(padc, (1, _W - P))], axis=1)

    nb = 4
    vi_sd = jax.ShapeDtypeStruct((num, 1, _W), jnp.int32)
    vf_sd = jax.ShapeDtypeStruct((num, 1, _W), jnp.float32)
    gidx, g, posf, ignf = pl.pallas_call(
        _match_kernel,
        grid=(num // nb,),
        in_specs=[
            pl.BlockSpec((4, _W), lambda i: (0, 0)),
            pl.BlockSpec((nb, n_obj, 5), lambda i: (i, 0, 0)),
        ],
        out_specs=[
            pl.BlockSpec((nb, 1, _W), lambda i: (i, 0, 0)),
            pl.BlockSpec((nb, 4, _W), lambda i: (i, 0, 0)),
            pl.BlockSpec((nb, 1, _W), lambda i: (i, 0, 0)),
            pl.BlockSpec((nb, 1, _W), lambda i: (i, 0, 0)),
        ],
        out_shape=[vi_sd,
                   jax.ShapeDtypeStruct((num, 4, _W), jnp.float32),
                   vf_sd, vf_sd],
        compiler_params=pltpu.CompilerParams(
            dimension_semantics=("arbitrary",),
        ),
    )(priors_p, targets)

    xg = _sc_gather(conf_t.reshape(-1), gidx.reshape(-1))

    total = num * P * num_classes
    sc_sd = jax.ShapeDtypeStruct((num, 1, 1), jnp.float32)
    s0, loc_sum, corr, pos_cnt, ign_cnt = pl.pallas_call(
        _loss_kernel,
        grid=(num // nb,),
        in_specs=[
            pl.BlockSpec((nb, num_classes, P), lambda i: (i, 0, 0)),
            pl.BlockSpec((nb, 4, P), lambda i: (i, 0, 0)),
            pl.BlockSpec((nb, 4, _W), lambda i: (i, 0, 0)),
            pl.BlockSpec((nb, 1, _W), lambda i: (i, 0, 0)),
            pl.BlockSpec((nb, 1, _W), lambda i: (i, 0, 0)),
            pl.BlockSpec((nb, 1, _W), lambda i: (i, 0, 0)),
        ],
        out_specs=[pl.BlockSpec((nb, 1, 1), lambda i: (i, 0, 0))] * 5,
        out_shape=[sc_sd] * 5,
        compiler_params=pltpu.CompilerParams(
            dimension_semantics=("arbitrary",),
        ),
    )(conf_t, loc_t, g, xg.reshape(num, 1, _W), posf, ignf)

    loss_l = jnp.sum(loc_sum) / (4.0 * jnp.sum(pos_cnt))
    denom = jnp.float32(total) - jnp.sum(ign_cnt)
    loss_c = (jnp.sum(s0) + jnp.sum(corr)) / denom
    return (loss_l, loss_c)
